# baseline (speedup 1.0000x reference)
import functools

import jax
import jax.numpy as jnp
from jax import lax
from jax.experimental import pallas as pl
from jax.experimental.pallas import tpu as pltpu
from jax.experimental.pallas import tpu_sc as plsc

_NC, _NS, _L = 2, 16, 16
_NW = _NC * _NS
_B = 16384
_PER_W = _B // _NW
_C = 128
_NCHUNK = _PER_W // _C
_D = 64
_MARGIN = 1.0


def _rsqrt(x):
    i = lax.bitcast_convert_type(x, jnp.int32)
    i = jnp.int32(0x5F3759DF) - lax.shift_right_arithmetic(i, 1)
    y = lax.bitcast_convert_type(i, jnp.float32)
    for _ in range(3):
        y = y * (1.5 - 0.5 * x * y * y)
    return y


def _sc_body(sp, tp, sn, tn, r, node, link, norm, out,
             isp, itp, isn, itn, ir,
             bsp, btp, bsn, btn, brm, bw, accv, sem):
    wid = lax.axis_index("s") * _NC + lax.axis_index("c")
    base = wid * _PER_W
    iota = lax.iota(jnp.int32, _L)
    acc = jnp.zeros((_L,), jnp.float32)
    for c in range(_NCHUNK):
        off = pl.multiple_of(base + c * _C, _C)
        pltpu.sync_copy(sp.at[pl.ds(off, _C)], isp)
        pltpu.sync_copy(tp.at[pl.ds(off, _C)], itp)
        pltpu.sync_copy(sn.at[pl.ds(off, _C)], isn)
        pltpu.sync_copy(tn.at[pl.ds(off, _C)], itn)
        pltpu.sync_copy(r.at[pl.ds(off, _C)], ir)
        cps = [
            pltpu.async_copy(node.at[isp], bsp, sem),
            pltpu.async_copy(node.at[itp], btp, sem),
            pltpu.async_copy(node.at[isn], bsn, sem),
            pltpu.async_copy(node.at[itn], btn, sem),
            pltpu.async_copy(link.at[ir], brm, sem),
            pltpu.async_copy(norm.at[ir], bw, sem),
        ]
        for cp in cps:
            cp.wait()

        def gbody(g, acc):
            rows = iota + g * _L

            def jbody(j, carry):
                qp, mp, qn, mn, ww, rw = carry
                cols = jnp.full((_L,), j, jnp.int32)
                vsp = plsc.load_gather(bsp, [rows, cols])
                vtp = plsc.load_gather(btp, [rows, cols])
                vsn = plsc.load_gather(bsn, [rows, cols])
                vtn = plsc.load_gather(btn, [rows, cols])
                vr = plsc.load_gather(brm, [rows, cols])
                vw = plsc.load_gather(bw, [rows, cols])
                bp = vsp - vtp + vr
                bn = vsn - vtn + vr
                return (qp + bp * bp, mp + bp * vw,
                        qn + bn * bn, mn + bn * vw,
                        ww + vw * vw, rw + vr * vw)

            z = jnp.zeros((_L,), jnp.float32)
            qp, mp, qn, mn, ww, rw = lax.fori_loop(
                0, _D, jbody, (z, z, z, z, z, z))
            cfp = (mp - rw) / ww
            cfn = (mn - rw) / ww
            ddp = qp - 2.0 * cfp * mp + cfp * cfp * ww
            ddn = qn - 2.0 * cfn * mn + cfn * cfn * ww
            ddp = jnp.maximum(ddp, 1e-20)
            ddn = jnp.maximum(ddn, 1e-20)
            dp = ddp * _rsqrt(ddp)
            dn = ddn * _rsqrt(ddn)
            return acc + jnp.maximum(0.0, dp - dn + _MARGIN)

        acc = lax.fori_loop(0, _C // _L, gbody, acc)
    accv[...] = acc
    pltpu.sync_copy(accv, out.at[wid])


_mesh = plsc.VectorSubcoreMesh(core_axis_name="c", subcore_axis_name="s")

_sc_kernel = pl.kernel(
    _sc_body,
    out_type=jax.ShapeDtypeStruct((_NW, _L), jnp.float32),
    mesh=_mesh,
    compiler_params=pltpu.CompilerParams(
        needs_layout_passes=False, use_tc_tiling_on_sc=False),
    scratch_types=[
        pltpu.VMEM((_C,), jnp.int32),
        pltpu.VMEM((_C,), jnp.int32),
        pltpu.VMEM((_C,), jnp.int32),
        pltpu.VMEM((_C,), jnp.int32),
        pltpu.VMEM((_C,), jnp.int32),
        pltpu.VMEM((_C, _D), jnp.float32),
        pltpu.VMEM((_C, _D), jnp.float32),
        pltpu.VMEM((_C, _D), jnp.float32),
        pltpu.VMEM((_C, _D), jnp.float32),
        pltpu.VMEM((_C, _D), jnp.float32),
        pltpu.VMEM((_C, _D), jnp.float32),
        pltpu.VMEM((_L,), jnp.float32),
        pltpu.SemaphoreType.DMA,
    ],
)


def kernel(sp, tp, sn, tn, r, node_emb, link_emb, norm_vector):
    sp = sp.astype(jnp.int32)
    tp = tp.astype(jnp.int32)
    sn = sn.astype(jnp.int32)
    tn = tn.astype(jnp.int32)
    r = r.astype(jnp.int32)
    partial = _sc_kernel(sp, tp, sn, tn, r, node_emb, link_emb, norm_vector)
    return jnp.sum(partial) / _B
